# baseline (speedup 1.0000x reference)
import functools

import jax
import jax.numpy as jnp
from jax import lax
from jax.experimental import pallas as pl
from jax.experimental.pallas import tpu as pltpu
from jax.experimental.pallas import tpu_sc as plsc

EMB = 64
MAX_NORM = EMB * 0.1
NC = 2
NS = 16
L = 16
NW = NC * NS


def _rsqrt_newton(x):
    i = lax.bitcast_convert_type(x, jnp.uint32)
    i = jnp.uint32(0x5F3759DF) - lax.shift_right_logical(i, jnp.uint32(1))
    y = lax.bitcast_convert_type(i, jnp.float32)
    half = jnp.float32(0.5) * x
    for _ in range(3):
        y = y * (jnp.float32(1.5) - half * y * y)
    return y


def _dssm_body(uid_hbm, nid_hbm, user_hbm, item_hbm, out_hbm,
               uidx_v, nidx_v, urows, irows, out_v, tbuf_u, tbuf_i, tbuf_d,
               sem_u, sem_i, bpw):
    wid = lax.axis_index("s") * NC + lax.axis_index("c")
    base = wid * bpw

    pltpu.sync_copy(uid_hbm.at[pl.ds(base, bpw)], uidx_v)
    pltpu.sync_copy(nid_hbm.at[pl.ds(base, bpw)], nidx_v)
    cu = pltpu.async_copy(user_hbm.at[uidx_v], urows, sem_u)
    ci = pltpu.async_copy(item_hbm.at[nidx_v], irows, sem_i)
    cu.wait()
    ci.wait()

    maxn = jnp.float32(MAX_NORM)
    eps = jnp.float32(1e-7)
    one = jnp.float32(1.0)
    lanes = lax.iota(jnp.int32, L)

    def grp(g, carry):
        for t in range(L):
            r = g * L + t
            u0 = urows[r, pl.ds(0, L)]
            u1 = urows[r, pl.ds(L, L)]
            u2 = urows[r, pl.ds(2 * L, L)]
            u3 = urows[r, pl.ds(3 * L, L)]
            i0 = irows[r, pl.ds(0, L)]
            i1 = irows[r, pl.ds(L, L)]
            i2 = irows[r, pl.ds(2 * L, L)]
            i3 = irows[r, pl.ds(3 * L, L)]
            su = (u0 * u0 + u1 * u1) + (u2 * u2 + u3 * u3)
            si = (i0 * i0 + i1 * i1) + (i2 * i2 + i3 * i3)
            sd = (u0 * i0 + u1 * i1) + (u2 * i2 + u3 * i3)
            col = jnp.full((L,), t, jnp.int32)
            plsc.store_scatter(tbuf_u, [lanes, col], su)
            plsc.store_scatter(tbuf_i, [lanes, col], si)
            plsc.store_scatter(tbuf_d, [lanes, col], sd)
        accu = tbuf_u[0, pl.ds(0, L)]
        acci = tbuf_i[0, pl.ds(0, L)]
        accd = tbuf_d[0, pl.ds(0, L)]
        for k in range(1, L):
            accu = accu + tbuf_u[k, pl.ds(0, L)]
            acci = acci + tbuf_i[k, pl.ds(0, L)]
            accd = accd + tbuf_d[k, pl.ds(0, L)]
        norm_u = accu * _rsqrt_newton(accu)
        norm_i = acci * _rsqrt_newton(acci)
        scale_u = jnp.minimum(one, maxn / (norm_u + eps))
        scale_i = jnp.minimum(one, maxn / (norm_i + eps))
        y = accd * (scale_u * scale_i)
        out_v[pl.ds(g * L, L)] = one / (one + jnp.exp(-y))
        return carry

    lax.fori_loop(0, bpw // L, grp, 0, unroll=False)

    pltpu.sync_copy(out_v, out_hbm.at[pl.ds(base, bpw)])


def kernel(uid, nid, user_emb, item_emb):
    b = uid.shape[0]
    bpw = b // NW
    mesh = plsc.VectorSubcoreMesh(core_axis_name="c", subcore_axis_name="s")
    k = functools.partial(
        pl.kernel,
        out_type=jax.ShapeDtypeStruct((b,), jnp.float32),
        mesh=mesh,
        compiler_params=pltpu.CompilerParams(
            needs_layout_passes=False, use_tc_tiling_on_sc=False),
        scratch_types=[
            pltpu.VMEM((bpw,), jnp.int32),
            pltpu.VMEM((bpw,), jnp.int32),
            pltpu.VMEM((bpw, EMB), jnp.float32),
            pltpu.VMEM((bpw, EMB), jnp.float32),
            pltpu.VMEM((bpw,), jnp.float32),
            pltpu.VMEM((L, L + 1), jnp.float32),
            pltpu.VMEM((L, L + 1), jnp.float32),
            pltpu.VMEM((L, L + 1), jnp.float32),
            pltpu.SemaphoreType.DMA,
            pltpu.SemaphoreType.DMA,
        ],
    )(functools.partial(_dssm_body, bpw=bpw))
    return k(uid.astype(jnp.int32), nid.astype(jnp.int32), user_emb, item_emb)


# speedup vs baseline: 2.2611x; 2.2611x over previous
import functools

import jax
import jax.numpy as jnp
from jax import lax
from jax.experimental import pallas as pl
from jax.experimental.pallas import tpu as pltpu
from jax.experimental.pallas import tpu_sc as plsc

EMB = 64
MAX_NORM = EMB * 0.1
NC = 2
NS = 16
L = 16
NW = NC * NS
SLAB = 8
CH = 64


def _rsqrt_newton(x):
    i = lax.bitcast_convert_type(x, jnp.uint32)
    i = jnp.uint32(0x5F3759DF) - lax.shift_right_logical(i, jnp.uint32(1))
    y = lax.bitcast_convert_type(i, jnp.float32)
    half = jnp.float32(0.5) * x
    for _ in range(3):
        y = y * (jnp.float32(1.5) - half * y * y)
    return y


def _dssm_body(uid_hbm, nid_hbm, user_hbm, item_hbm, out_hbm,
               uidx_v, nidx_v, ubuf, ibuf, out_v,
               tbuf_u, tbuf_i, tbuf_d, sem_u, sem_i, bpw):
    wid = lax.axis_index("s") * NC + lax.axis_index("c")
    base = wid * bpw

    pltpu.sync_copy(uid_hbm.at[pl.ds(base, bpw)], uidx_v)
    pltpu.sync_copy(nid_hbm.at[pl.ds(base, bpw)], nidx_v)

    maxn = jnp.float32(MAX_NORM)
    eps = jnp.float32(1e-7)
    one = jnp.float32(1.0)
    lanes = lax.iota(jnp.int32, L)

    def chunk(k, carry):
        ucopies = []
        icopies = []
        for g in range(CH // L):
            uvec = uidx_v[pl.ds(k * CH + g * L, L)]
            nvec = nidx_v[pl.ds(k * CH + g * L, L)]
            ut = lax.shift_right_logical(uvec, 3)
            us = uvec & 7
            nt = lax.shift_right_logical(nvec, 3)
            ns = nvec & 7
            for t in range(L):
                j = g * L + t
                ucopies.append(pltpu.make_async_copy(
                    user_hbm.at[ut[t], us[t]], ubuf.at[j], sem_u))
                icopies.append(pltpu.make_async_copy(
                    item_hbm.at[nt[t], ns[t]], ibuf.at[j], sem_i))
        for c in ucopies:
            c.start()
        for c in icopies:
            c.start()
        for c in ucopies:
            c.wait()
        for c in icopies:
            c.wait()

        def grp(g, carry2):
            for t in range(L):
                j = g * L + t
                u0 = ubuf[j, pl.ds(0, L)]
                u1 = ubuf[j, pl.ds(L, L)]
                u2 = ubuf[j, pl.ds(2 * L, L)]
                u3 = ubuf[j, pl.ds(3 * L, L)]
                i0 = ibuf[j, pl.ds(0, L)]
                i1 = ibuf[j, pl.ds(L, L)]
                i2 = ibuf[j, pl.ds(2 * L, L)]
                i3 = ibuf[j, pl.ds(3 * L, L)]
                su = (u0 * u0 + u1 * u1) + (u2 * u2 + u3 * u3)
                si = (i0 * i0 + i1 * i1) + (i2 * i2 + i3 * i3)
                sd = (u0 * i0 + u1 * i1) + (u2 * i2 + u3 * i3)
                col = jnp.full((L,), t, jnp.int32)
                plsc.store_scatter(tbuf_u, [lanes, col], su)
                plsc.store_scatter(tbuf_i, [lanes, col], si)
                plsc.store_scatter(tbuf_d, [lanes, col], sd)
            accu = tbuf_u[0, pl.ds(0, L)]
            acci = tbuf_i[0, pl.ds(0, L)]
            accd = tbuf_d[0, pl.ds(0, L)]
            for t in range(1, L):
                accu = accu + tbuf_u[t, pl.ds(0, L)]
                acci = acci + tbuf_i[t, pl.ds(0, L)]
                accd = accd + tbuf_d[t, pl.ds(0, L)]
            norm_u = accu * _rsqrt_newton(accu)
            norm_i = acci * _rsqrt_newton(acci)
            scale_u = jnp.minimum(one, maxn / (norm_u + eps))
            scale_i = jnp.minimum(one, maxn / (norm_i + eps))
            y = accd * (scale_u * scale_i)
            out_v[pl.ds(k * CH + g * L, L)] = one / (one + jnp.exp(-y))
            return carry2

        lax.fori_loop(0, CH // L, grp, 0, unroll=False)
        return carry

    lax.fori_loop(0, bpw // CH, chunk, 0, unroll=False)

    pltpu.sync_copy(out_v, out_hbm.at[pl.ds(base, bpw)])


def kernel(uid, nid, user_emb, item_emb):
    b = uid.shape[0]
    bpw = b // NW
    user3 = user_emb.reshape(user_emb.shape[0] // SLAB, SLAB, EMB)
    item3 = item_emb.reshape(item_emb.shape[0] // SLAB, SLAB, EMB)
    mesh = plsc.VectorSubcoreMesh(core_axis_name="c", subcore_axis_name="s")
    k = functools.partial(
        pl.kernel,
        out_type=jax.ShapeDtypeStruct((b,), jnp.float32),
        mesh=mesh,
        compiler_params=pltpu.CompilerParams(needs_layout_passes=False),
        scratch_types=[
            pltpu.VMEM((bpw,), jnp.int32),
            pltpu.VMEM((bpw,), jnp.int32),
            pltpu.VMEM((CH, EMB), jnp.float32),
            pltpu.VMEM((CH, EMB), jnp.float32),
            pltpu.VMEM((bpw,), jnp.float32),
            pltpu.VMEM((L, L + 1), jnp.float32),
            pltpu.VMEM((L, L + 1), jnp.float32),
            pltpu.VMEM((L, L + 1), jnp.float32),
            pltpu.SemaphoreType.DMA,
            pltpu.SemaphoreType.DMA,
        ],
    )(functools.partial(_dssm_body, bpw=bpw))
    return k(uid.astype(jnp.int32), nid.astype(jnp.int32), user3, item3)
